# baseline (speedup 1.0000x reference)
import functools

import jax
import jax.numpy as jnp
from jax import lax
from jax.experimental import pallas as pl
from jax.experimental.pallas import tpu as pltpu
from jax.experimental.pallas import tpu_sc as plsc

_N = 10000
_D = 128
_H = 128
_C = 10

_NP = 10240
_NPS = 10000
_K = 56
_RING = 2
_NW = 32
_CHUNKS = 184
_IBLK = 8
_NBLK = _CHUNKS // _IBLK
_EPT = _CHUNKS * _K
_EP = _NW * _EPT
_RPTA = 632
_RPTB = _NPS - 15 * _RPTA


def _tables_body(x_ref, w1_ref, w2_ref, be_ref, xa_ref, bt_ref):
    xb = x_ref[...]
    xa_ref[:, 0:_D] = xb
    xa_ref[:, _D:2 * _D] = jnp.dot(xb, w1_ref[...],
                                   preferred_element_type=jnp.float32)
    bt_ref[...] = jnp.dot(xb, w2_ref[...],
                          preferred_element_type=jnp.float32) + be_ref[...]


def _make_tables(xp, w1, w2, be):
    blk = 1024
    grid = _NP // blk
    return pl.pallas_call(
        _tables_body,
        grid=(grid,),
        in_specs=[
            pl.BlockSpec((blk, _D), lambda i: (i, 0)),
            pl.BlockSpec((_D, _H), lambda i: (0, 0)),
            pl.BlockSpec((_D, _H), lambda i: (0, 0)),
            pl.BlockSpec((1, _H), lambda i: (0, 0)),
        ],
        out_specs=[
            pl.BlockSpec((blk, 2 * _D), lambda i: (i, 0)),
            pl.BlockSpec((blk, _H), lambda i: (i, 0)),
        ],
        out_shape=[
            jax.ShapeDtypeStruct((_NP, 2 * _D), jnp.float32),
            jax.ShapeDtypeStruct((_NP, _H), jnp.float32),
        ],
    )(xp, w1, w2, be)


def _sc_body(xa_hbm, bt_hbm, src_hbm, dst_hbm, out_hbm,
             src_i, dst_i, xa0, xa1, b0, b1, hneigh,
             sx0, sx1, sb0, sb1, ss0, ss1):
    c = lax.axis_index("c")
    s = lax.axis_index("s")
    wid = c * 16 + s
    rbase = wid * _CHUNKS

    bufs = ((xa0, b0, sx0, sb0, ss0),
            (xa1, b1, sx1, sb1, ss1))

    def _zrow(r, carry):
        for cc in range(8):
            b0[r, pl.ds(cc * 16, 16)] = jnp.zeros((16,), jnp.float32)
        return carry
    lax.fori_loop(0, _K, _zrow, 0)

    def _span(body):
        @pl.when(s < 15)
        def _():
            row0 = pl.multiple_of(s * _RPTA, 8)
            for j in range(_RPTA // _K):
                body(row0 + j * _K, _K)
            body(row0 + (_RPTA // _K) * _K, _RPTA % _K)

        @pl.when(s == 15)
        def _():
            row0 = 15 * _RPTA
            for j in range(_RPTB // _K):
                body(row0 + j * _K, _K)
            body(row0 + (_RPTB // _K) * _K, _RPTB % _K)

    def _zcopy(r0, n):
        pltpu.sync_copy(b0.at[pl.ds(0, n)], hneigh.at[pl.ds(r0, n)])
    _span(_zcopy)
    plsc.subcore_barrier()

    def _load_iblock(b):
        r0 = pl.multiple_of(rbase + b * _IBLK, 8)
        q = b % 2
        pltpu.sync_copy(src_hbm.at[pl.ds(r0, _IBLK)], src_i.at[q])
        pltpu.sync_copy(dst_hbm.at[pl.ds(r0, _IBLK)], dst_i.at[q])

    def _issue(i, p):
        xa, bb, sx, sb, ss = bufs[p]
        q = (i // _IBLK) % 2
        j = i % _IBLK
        pltpu.async_copy(xa_hbm.at[src_i.at[q, j]], xa, sx)
        pltpu.async_copy(bt_hbm.at[dst_i.at[q, j]], bb, sb)

    def _process(i, p):
        xa, bb, sx, sb, ss = bufs[p]
        po = 1 - p
        xao, bbo, sxo, sbo, sso = bufs[po]
        q = (i // _IBLK) % 2
        j = i % _IBLK

        @pl.when(i >= 1)
        def _():
            pltpu.make_async_copy(bbo, hneigh.at[dst_i.at[q, j]], sso).wait()

        @pl.when((i % _IBLK == 0) & (i // _IBLK + 1 < _NBLK))
        def _():
            _load_iblock(i // _IBLK + 1)

        @pl.when(i + 1 < _CHUNKS)
        def _():
            _issue(i + 1, po)

        pltpu.make_async_copy(xa_hbm.at[src_i.at[q, j]], xa, sx).wait()
        pltpu.make_async_copy(bt_hbm.at[dst_i.at[q, j]], bb, sb).wait()

        @plsc.parallel_loop(0, _K, unroll=2)
        def _crow(r):
            for cc in range(8):
                xv = xa[r, pl.ds(cc * 16, 16)]
                av = xa[r, pl.ds(_D + cc * 16, 16)]
                bv = bb[r, pl.ds(cc * 16, 16)]
                bb[r, pl.ds(cc * 16, 16)] = xv * jnp.maximum(av + bv, 0.0)

        pltpu.async_copy(bb, hneigh.at[dst_i.at[q, j]], ss, add=True)

    _load_iblock(0)
    _issue(0, 0)

    def _step(k, carry):
        _process(2 * k, 0)
        _process(2 * k + 1, 1)
        return carry
    lax.fori_loop(0, _CHUNKS // 2, _step, 0)

    lastb = bufs[(_CHUNKS - 1) % _RING]
    pltpu.make_async_copy(lastb[1], hneigh.at[dst_i.at[0, 0]],
                          lastb[4]).wait()

    plsc.subcore_barrier()

    def _dcopy(r0, n):
        pltpu.sync_copy(hneigh.at[pl.ds(r0, n)], b0.at[pl.ds(0, n)])
        pltpu.sync_copy(b0.at[pl.ds(0, n)], out_hbm.at[c, pl.ds(r0, n)])
    _span(_dcopy)


_sc_edges = functools.partial(
    pl.kernel,
    out_type=jax.ShapeDtypeStruct((2, _NPS, _H), jnp.float32),
    mesh=plsc.VectorSubcoreMesh(core_axis_name="c", subcore_axis_name="s"),
    scratch_types=(
        [pltpu.VMEM((2, _IBLK, _K), jnp.int32)] * 2
        + [pltpu.VMEM((_K, 2 * _D), jnp.float32)] * _RING
        + [pltpu.VMEM((_K, _H), jnp.float32)] * _RING
        + [pltpu.VMEM_SHARED((_NPS, _H), jnp.float32)]
        + [pltpu.SemaphoreType.DMA] * (3 * _RING)
    ),
)(_sc_body)


def _readout_body(p0_ref, p1_ref, wn_ref, bn_ref, wg_ref, bg_ref,
                  wf_ref, bf_ref, out_ref, m_ref, s_ref, r_ref):
    step = pl.program_id(0)

    @pl.when(step == 0)
    def _():
        m_ref[0, 0] = -1e30
        s_ref[0, 0] = 0.0
        r_ref[...] = jnp.zeros_like(r_ref)

    p = p0_ref[...] + p1_ref[...]
    h = jnp.maximum(
        jnp.dot(p, wn_ref[...], preferred_element_type=jnp.float32)
        + bn_ref[...], 0.0)
    g = jnp.dot(h, wg_ref[...], preferred_element_type=jnp.float32) + bg_ref[0, 0]
    rows = step * p.shape[0] + lax.broadcasted_iota(jnp.int32, g.shape, 0)
    g = jnp.where(rows < _N, g, -1e30)

    m_old = m_ref[0, 0]
    m_new = jnp.maximum(m_old, jnp.max(g))
    scale = jnp.exp(m_old - m_new)
    e = jnp.exp(g - m_new)
    s_ref[0, 0] = s_ref[0, 0] * scale + jnp.sum(e)
    r_ref[...] = r_ref[...] * scale + jnp.sum(e * h, axis=0, keepdims=True)
    m_ref[0, 0] = m_new

    @pl.when(step == pl.num_programs(0) - 1)
    def _():
        readout = r_ref[...] / s_ref[0, 0]
        out_ref[...] = jnp.dot(readout, wf_ref[...],
                               preferred_element_type=jnp.float32) + bf_ref[...]


def _readout(p0, p1, wn, bn, wg, bg, wf, bf):
    blk = 1024
    grid = _NP // blk
    return pl.pallas_call(
        _readout_body,
        grid=(grid,),
        in_specs=[
            pl.BlockSpec((blk, _H), lambda i: (i, 0)),
            pl.BlockSpec((blk, _H), lambda i: (i, 0)),
            pl.BlockSpec((_H, _H), lambda i: (0, 0)),
            pl.BlockSpec((1, _H), lambda i: (0, 0)),
            pl.BlockSpec((_H, 1), lambda i: (0, 0)),
            pl.BlockSpec((1, 1), lambda i: (0, 0)),
            pl.BlockSpec((_H, _C), lambda i: (0, 0)),
            pl.BlockSpec((1, _C), lambda i: (0, 0)),
        ],
        out_specs=pl.BlockSpec((1, _C), lambda i: (0, 0)),
        out_shape=jax.ShapeDtypeStruct((1, _C), jnp.float32),
        scratch_shapes=[
            pltpu.SMEM((1, 1), jnp.float32),
            pltpu.SMEM((1, 1), jnp.float32),
            pltpu.VMEM((1, _H), jnp.float32),
        ],
    )(p0, p1, wn, bn, wg, bg, wf, bf)


def kernel(x, edge_index, W_edge, b_edge, W_node, b_node, W_gate, b_gate,
           W_fc, b_fc):
    xp = jnp.pad(x, ((0, _NP - _N), (0, 0)))
    xa, bt = _make_tables(xp, W_edge[:_D], W_edge[_D:],
                          b_edge.reshape(1, _H))

    e = edge_index.shape[1]
    src = jnp.concatenate(
        [edge_index[0], jnp.full((_EP - e,), _N, jnp.int32)]
    ).reshape(_NW * _CHUNKS, _K)
    dump = jnp.arange(_EP - e, dtype=jnp.int32) % _N
    dst = jnp.concatenate([edge_index[1], dump]).reshape(_NW * _CHUNKS, _K)

    partials = _sc_edges(xa, bt, src, dst)
    partials = jnp.pad(partials, ((0, 0), (0, _NP - _NPS), (0, 0)))

    return _readout(partials[0], partials[1], W_node,
                    b_node.reshape(1, _H), W_gate, b_gate.reshape(1, 1),
                    W_fc, b_fc.reshape(1, _C))


# speedup vs baseline: 1.2232x; 1.2232x over previous
import functools

import jax
import jax.numpy as jnp
from jax import lax
from jax.experimental import pallas as pl
from jax.experimental.pallas import tpu as pltpu
from jax.experimental.pallas import tpu_sc as plsc

_N = 10000
_D = 128
_H = 128
_C = 10

_NP = 10240
_NPS = 10000
_K = 40
_RING = 3
_NW = 32
_CHUNKS = 256
_IBLK = 8
_NBLK = _CHUNKS // _IBLK
_EPT = _CHUNKS * _K
_EP = _NW * _EPT
_RPTA = 632
_RPTB = _NPS - 15 * _RPTA


def _tables_body(x_ref, w1_ref, w2_ref, be_ref, xa_ref, bt_ref):
    xb = x_ref[...]
    xa_ref[:, 0:_D] = xb
    xa_ref[:, _D:2 * _D] = jnp.dot(xb, w1_ref[...],
                                   preferred_element_type=jnp.float32)
    bt_ref[...] = jnp.dot(xb, w2_ref[...],
                          preferred_element_type=jnp.float32) + be_ref[...]


def _make_tables(xp, w1, w2, be):
    blk = 1024
    grid = _NP // blk
    return pl.pallas_call(
        _tables_body,
        grid=(grid,),
        in_specs=[
            pl.BlockSpec((blk, _D), lambda i: (i, 0)),
            pl.BlockSpec((_D, _H), lambda i: (0, 0)),
            pl.BlockSpec((_D, _H), lambda i: (0, 0)),
            pl.BlockSpec((1, _H), lambda i: (0, 0)),
        ],
        out_specs=[
            pl.BlockSpec((blk, 2 * _D), lambda i: (i, 0)),
            pl.BlockSpec((blk, _H), lambda i: (i, 0)),
        ],
        out_shape=[
            jax.ShapeDtypeStruct((_NP, 2 * _D), jnp.float32),
            jax.ShapeDtypeStruct((_NP, _H), jnp.float32),
        ],
    )(xp, w1, w2, be)


def _sc_body(xa_hbm, bt_hbm, src_hbm, dst_hbm, out_hbm,
             src_i, dst_i, xa0, xa1, xa2, b0, b1, b2, hneigh,
             sx0, sx1, sx2, sb0, sb1, sb2, ss0, ss1, ss2):
    c = lax.axis_index("c")
    s = lax.axis_index("s")
    wid = c * 16 + s
    rbase = wid * _CHUNKS

    bufs = ((xa0, b0, sx0, sb0, ss0),
            (xa1, b1, sx1, sb1, ss1),
            (xa2, b2, sx2, sb2, ss2))

    def _zrow(r, carry):
        for cc in range(8):
            b0[r, pl.ds(cc * 16, 16)] = jnp.zeros((16,), jnp.float32)
        return carry
    lax.fori_loop(0, _K, _zrow, 0)

    def _span(body):
        @pl.when(s < 15)
        def _():
            row0 = pl.multiple_of(s * _RPTA, 8)
            for j in range(_RPTA // _K):
                body(row0 + j * _K, _K)
            body(row0 + (_RPTA // _K) * _K, _RPTA % _K)

        @pl.when(s == 15)
        def _():
            row0 = 15 * _RPTA
            for j in range(_RPTB // _K):
                body(row0 + j * _K, _K)

    def _zcopy(r0, n):
        pltpu.sync_copy(b0.at[pl.ds(0, n)], hneigh.at[pl.ds(r0, n)])
    _span(_zcopy)
    plsc.subcore_barrier()

    def _load_iblock(b):
        r0 = pl.multiple_of(rbase + b * _IBLK, 8)
        q = b % 2
        pltpu.sync_copy(src_hbm.at[pl.ds(r0, _IBLK)], src_i.at[q])
        pltpu.sync_copy(dst_hbm.at[pl.ds(r0, _IBLK)], dst_i.at[q])

    def _issue(i, p):
        xa, bb, sx, sb, ss = bufs[p]
        q = (i // _IBLK) % 2
        j = i % _IBLK
        pltpu.async_copy(xa_hbm.at[src_i.at[q, j]], xa, sx)
        pltpu.async_copy(bt_hbm.at[dst_i.at[q, j]], bb, sb)

    def _process(i, p):
        xa, bb, sx, sb, ss = bufs[p]
        q = (i // _IBLK) % 2
        j = i % _IBLK

        pltpu.make_async_copy(xa_hbm.at[src_i.at[q, j]], xa, sx).wait()
        pltpu.make_async_copy(bt_hbm.at[dst_i.at[q, j]], bb, sb).wait()

        @plsc.parallel_loop(0, _K, unroll=2)
        def _crow(r):
            for cc in range(8):
                xv = xa[r, pl.ds(cc * 16, 16)]
                av = xa[r, pl.ds(_D + cc * 16, 16)]
                bv = bb[r, pl.ds(cc * 16, 16)]
                bb[r, pl.ds(cc * 16, 16)] = xv * jnp.maximum(av + bv, 0.0)

        pltpu.async_copy(bb, hneigh.at[dst_i.at[q, j]], ss, add=True)

        pn = (p + 2) % _RING
        bbn = bufs[pn][1]
        ssn = bufs[pn][4]

        @pl.when(i >= 1)
        def _():
            pltpu.make_async_copy(bbn, hneigh.at[dst_i.at[q, j]], ssn).wait()

        @pl.when((i % _IBLK == 0) & (i // _IBLK + 1 < _NBLK))
        def _():
            _load_iblock(i // _IBLK + 1)

        @pl.when(i + 2 < _CHUNKS)
        def _():
            _issue(i + 2, pn)

    _load_iblock(0)
    _issue(0, 0)
    _issue(1, 1)

    def _step(k, carry):
        _process(3 * k, 0)
        _process(3 * k + 1, 1)
        _process(3 * k + 2, 2)
        return carry
    lax.fori_loop(0, _CHUNKS // 3, _step, 0)
    _process(jnp.int32(_CHUNKS - 1), (_CHUNKS - 1) % _RING)

    lastb = bufs[(_CHUNKS - 1) % _RING]
    pltpu.make_async_copy(lastb[1], hneigh.at[dst_i.at[0, 0]],
                          lastb[4]).wait()

    plsc.subcore_barrier()

    def _dcopy(r0, n):
        pltpu.sync_copy(hneigh.at[pl.ds(r0, n)], b0.at[pl.ds(0, n)])
        pltpu.sync_copy(b0.at[pl.ds(0, n)], out_hbm.at[c, pl.ds(r0, n)])
    _span(_dcopy)


_sc_edges = functools.partial(
    pl.kernel,
    out_type=jax.ShapeDtypeStruct((2, _NPS, _H), jnp.float32),
    mesh=plsc.VectorSubcoreMesh(core_axis_name="c", subcore_axis_name="s"),
    scratch_types=(
        [pltpu.VMEM((2, _IBLK, _K), jnp.int32)] * 2
        + [pltpu.VMEM((_K, 2 * _D), jnp.float32)] * _RING
        + [pltpu.VMEM((_K, _H), jnp.float32)] * _RING
        + [pltpu.VMEM_SHARED((_NPS, _H), jnp.float32)]
        + [pltpu.SemaphoreType.DMA] * (3 * _RING)
    ),
)(_sc_body)


def _readout_body(p0_ref, p1_ref, wn_ref, bn_ref, wg_ref, bg_ref,
                  wf_ref, bf_ref, out_ref, m_ref, s_ref, r_ref):
    step = pl.program_id(0)

    @pl.when(step == 0)
    def _():
        m_ref[0, 0] = -1e30
        s_ref[0, 0] = 0.0
        r_ref[...] = jnp.zeros_like(r_ref)

    p = p0_ref[...] + p1_ref[...]
    h = jnp.maximum(
        jnp.dot(p, wn_ref[...], preferred_element_type=jnp.float32)
        + bn_ref[...], 0.0)
    g = jnp.dot(h, wg_ref[...], preferred_element_type=jnp.float32) + bg_ref[0, 0]
    rows = step * p.shape[0] + lax.broadcasted_iota(jnp.int32, g.shape, 0)
    g = jnp.where(rows < _N, g, -1e30)

    m_old = m_ref[0, 0]
    m_new = jnp.maximum(m_old, jnp.max(g))
    scale = jnp.exp(m_old - m_new)
    e = jnp.exp(g - m_new)
    s_ref[0, 0] = s_ref[0, 0] * scale + jnp.sum(e)
    r_ref[...] = r_ref[...] * scale + jnp.sum(e * h, axis=0, keepdims=True)
    m_ref[0, 0] = m_new

    @pl.when(step == pl.num_programs(0) - 1)
    def _():
        readout = r_ref[...] / s_ref[0, 0]
        out_ref[...] = jnp.dot(readout, wf_ref[...],
                               preferred_element_type=jnp.float32) + bf_ref[...]


def _readout(p0, p1, wn, bn, wg, bg, wf, bf):
    blk = 1024
    grid = _NP // blk
    return pl.pallas_call(
        _readout_body,
        grid=(grid,),
        in_specs=[
            pl.BlockSpec((blk, _H), lambda i: (i, 0)),
            pl.BlockSpec((blk, _H), lambda i: (i, 0)),
            pl.BlockSpec((_H, _H), lambda i: (0, 0)),
            pl.BlockSpec((1, _H), lambda i: (0, 0)),
            pl.BlockSpec((_H, 1), lambda i: (0, 0)),
            pl.BlockSpec((1, 1), lambda i: (0, 0)),
            pl.BlockSpec((_H, _C), lambda i: (0, 0)),
            pl.BlockSpec((1, _C), lambda i: (0, 0)),
        ],
        out_specs=pl.BlockSpec((1, _C), lambda i: (0, 0)),
        out_shape=jax.ShapeDtypeStruct((1, _C), jnp.float32),
        scratch_shapes=[
            pltpu.SMEM((1, 1), jnp.float32),
            pltpu.SMEM((1, 1), jnp.float32),
            pltpu.VMEM((1, _H), jnp.float32),
        ],
    )(p0, p1, wn, bn, wg, bg, wf, bf)


def kernel(x, edge_index, W_edge, b_edge, W_node, b_node, W_gate, b_gate,
           W_fc, b_fc):
    xp = jnp.pad(x, ((0, _NP - _N), (0, 0)))
    xa, bt = _make_tables(xp, W_edge[:_D], W_edge[_D:],
                          b_edge.reshape(1, _H))

    e = edge_index.shape[1]
    src = jnp.concatenate(
        [edge_index[0], jnp.full((_EP - e,), _N, jnp.int32)]
    ).reshape(_NW * _CHUNKS, _K)
    dump = jnp.arange(_EP - e, dtype=jnp.int32) % _N
    dst = jnp.concatenate([edge_index[1], dump]).reshape(_NW * _CHUNKS, _K)

    partials = _sc_edges(xa, bt, src, dst)
    partials = jnp.pad(partials, ((0, 0), (0, _NP - _NPS), (0, 0)))

    return _readout(partials[0], partials[1], W_node,
                    b_node.reshape(1, _H), W_gate, b_gate.reshape(1, 1),
                    W_fc, b_fc.reshape(1, _C))


# speedup vs baseline: 2.1372x; 1.7472x over previous
import functools

import jax
import jax.numpy as jnp
from jax import lax
from jax.experimental import pallas as pl
from jax.experimental.pallas import tpu as pltpu
from jax.experimental.pallas import tpu_sc as plsc

_N = 10000
_D = 128
_H = 128
_C = 10

_NP = 10240
_NPS = 10000
_K = 40
_RING = 3
_NW = 32
_CHUNKS = 256
_IBLK = 8
_NBLK = _CHUNKS // _IBLK
_EPT = _CHUNKS * _K
_EP = _NW * _EPT
_RPTA = 632
_RPTB = _NPS - 15 * _RPTA


def _tables_body(x_ref, w1_ref, w2_ref, be_ref, xa_ref, bt_ref):
    xb = x_ref[...]
    xa_ref[:, 0:_D] = xb
    xa_ref[:, _D:2 * _D] = jnp.dot(xb, w1_ref[...],
                                   preferred_element_type=jnp.float32)
    bt_ref[...] = jnp.dot(xb, w2_ref[...],
                          preferred_element_type=jnp.float32) + be_ref[...]


def _make_tables(xp, w1, w2, be):
    blk = 1024
    grid = _NP // blk
    return pl.pallas_call(
        _tables_body,
        grid=(grid,),
        in_specs=[
            pl.BlockSpec((blk, _D), lambda i: (i, 0)),
            pl.BlockSpec((_D, _H), lambda i: (0, 0)),
            pl.BlockSpec((_D, _H), lambda i: (0, 0)),
            pl.BlockSpec((1, _H), lambda i: (0, 0)),
        ],
        out_specs=[
            pl.BlockSpec((blk, 2 * _D), lambda i: (i, 0)),
            pl.BlockSpec((blk, _H), lambda i: (i, 0)),
        ],
        out_shape=[
            jax.ShapeDtypeStruct((_NP, 2 * _D), jnp.float32),
            jax.ShapeDtypeStruct((_NP, _H), jnp.float32),
        ],
    )(xp, w1, w2, be)


def _sc_body(xa_hbm, bt_hbm, src_hbm, dst_hbm, out_hbm,
             src_i, dst_i, xa0, xa1, xa2, b0, b1, b2, hneigh,
             sx0, sx1, sx2, sb0, sb1, sb2, ss0, ss1, ss2):
    c = lax.axis_index("c")
    s = lax.axis_index("s")
    wid = c * 16 + s
    rbase = wid * _CHUNKS

    bufs = ((xa0, b0, sx0, sb0, ss0),
            (xa1, b1, sx1, sb1, ss1),
            (xa2, b2, sx2, sb2, ss2))

    def _zrow(r, carry):
        for cc in range(8):
            b0[r, pl.ds(cc * 16, 16)] = jnp.zeros((16,), jnp.float32)
        return carry
    lax.fori_loop(0, _K, _zrow, 0)

    def _span(body):
        @pl.when(s < 15)
        def _():
            row0 = pl.multiple_of(s * _RPTA, 8)
            for j in range(_RPTA // _K):
                body(row0 + j * _K, _K)
            body(row0 + (_RPTA // _K) * _K, _RPTA % _K)

        @pl.when(s == 15)
        def _():
            row0 = 15 * _RPTA
            for j in range(_RPTB // _K):
                body(row0 + j * _K, _K)

    def _zcopy(r0, n):
        pltpu.sync_copy(b0.at[pl.ds(0, n)], hneigh.at[pl.ds(r0, n)])
    _span(_zcopy)
    plsc.subcore_barrier()

    def _load_iblock(b):
        r0 = pl.multiple_of(rbase + b * _IBLK, 8)
        q = b % 2
        pltpu.sync_copy(src_hbm.at[pl.ds(r0, _IBLK)], src_i.at[q])
        pltpu.sync_copy(dst_hbm.at[pl.ds(r0, _IBLK)], dst_i.at[q])

    def _issue(i, p):
        xa, bb, sx, sb, ss = bufs[p]
        q = (i // _IBLK) % 2
        j = i % _IBLK
        pltpu.async_copy(xa_hbm.at[src_i.at[q, j]], xa, sx)
        pltpu.async_copy(bt_hbm.at[dst_i.at[q, j]], bb, sb)

    def _process(i, p):
        xa, bb, sx, sb, ss = bufs[p]
        q = (i // _IBLK) % 2
        j = i % _IBLK

        pltpu.make_async_copy(xa_hbm.at[src_i.at[q, j]], xa, sx).wait()
        pltpu.make_async_copy(bt_hbm.at[dst_i.at[q, j]], bb, sb).wait()

        @plsc.parallel_loop(0, _K, unroll=2)
        def _crow(r):
            for cc in range(8):
                xv = xa[r, pl.ds(cc * 16, 16)]
                av = xa[r, pl.ds(_D + cc * 16, 16)]
                bv = bb[r, pl.ds(cc * 16, 16)]
                bb[r, pl.ds(cc * 16, 16)] = xv * jnp.maximum(av + bv, 0.0)

        pltpu.async_copy(bb, hneigh.at[dst_i.at[q, j]], ss, add=True)

        pn = (p + 2) % _RING
        bbn = bufs[pn][1]
        ssn = bufs[pn][4]

        @pl.when(i >= 1)
        def _():
            pltpu.make_async_copy(bbn, hneigh.at[dst_i.at[q, j]], ssn).wait()

        @pl.when((i % _IBLK == 0) & (i // _IBLK + 1 < _NBLK))
        def _():
            _load_iblock(i // _IBLK + 1)

        @pl.when(i + 2 < _CHUNKS)
        def _():
            _issue(i + 2, pn)

    _load_iblock(0)
    _issue(0, 0)
    _issue(1, 1)

    def _step(k, carry):
        _process(3 * k, 0)
        _process(3 * k + 1, 1)
        _process(3 * k + 2, 2)
        return carry
    lax.fori_loop(0, _CHUNKS // 3, _step, 0)
    _process(jnp.int32(_CHUNKS - 1), (_CHUNKS - 1) % _RING)

    lastb = bufs[(_CHUNKS - 1) % _RING]
    pltpu.make_async_copy(lastb[1], hneigh.at[dst_i.at[0, 0]],
                          lastb[4]).wait()

    plsc.subcore_barrier()

    def _dcopy(r0, n):
        pltpu.sync_copy(hneigh.at[pl.ds(r0, n)], b0.at[pl.ds(0, n)])
        pltpu.sync_copy(b0.at[pl.ds(0, n)], out_hbm.at[c, pl.ds(r0, n)])
    _span(_dcopy)


_sc_edges = functools.partial(
    pl.kernel,
    out_type=jax.ShapeDtypeStruct((2, _NPS, _H), jnp.float32),
    mesh=plsc.VectorSubcoreMesh(core_axis_name="c", subcore_axis_name="s"),
    scratch_types=(
        [pltpu.VMEM((2, _IBLK, _K), jnp.int32)] * 2
        + [pltpu.VMEM((_K, 2 * _D), jnp.float32)] * _RING
        + [pltpu.VMEM((_K, _H), jnp.float32)] * _RING
        + [pltpu.VMEM_SHARED((_NPS, _H), jnp.float32)]
        + [pltpu.SemaphoreType.DMA] * (3 * _RING)
    ),
)(_sc_body)


def _readout_body(p0_ref, p1_ref, wn_ref, bn_ref, wg_ref, bg_ref,
                  wf_ref, bf_ref, out_ref, m_ref, s_ref, r_ref):
    step = pl.program_id(0)

    @pl.when(step == 0)
    def _():
        m_ref[0, 0] = -1e30
        s_ref[0, 0] = 0.0
        r_ref[...] = jnp.zeros_like(r_ref)

    p = p0_ref[...] + p1_ref[...]
    h = jnp.maximum(
        jnp.dot(p, wn_ref[...], preferred_element_type=jnp.float32)
        + bn_ref[...], 0.0)
    g = jnp.dot(h, wg_ref[...], preferred_element_type=jnp.float32) + bg_ref[0, 0]
    rows = step * p.shape[0] + lax.broadcasted_iota(jnp.int32, g.shape, 0)
    g = jnp.where(rows < _N, g, -1e30)

    m_old = m_ref[0, 0]
    m_new = jnp.maximum(m_old, jnp.max(g))
    scale = jnp.exp(m_old - m_new)
    e = jnp.exp(g - m_new)
    s_ref[0, 0] = s_ref[0, 0] * scale + jnp.sum(e)
    r_ref[...] = r_ref[...] * scale + jnp.sum(e * h, axis=0, keepdims=True)
    m_ref[0, 0] = m_new

    @pl.when(step == pl.num_programs(0) - 1)
    def _():
        readout = r_ref[...] / s_ref[0, 0]
        out_ref[...] = jnp.dot(readout, wf_ref[...],
                               preferred_element_type=jnp.float32) + bf_ref[...]


def _readout(p0, p1, wn, bn, wg, bg, wf, bf):
    blk = 1024
    grid = _NP // blk
    return pl.pallas_call(
        _readout_body,
        grid=(grid,),
        in_specs=[
            pl.BlockSpec((blk, _H), lambda i: (i, 0)),
            pl.BlockSpec((blk, _H), lambda i: (i, 0)),
            pl.BlockSpec((_H, _H), lambda i: (0, 0)),
            pl.BlockSpec((1, _H), lambda i: (0, 0)),
            pl.BlockSpec((_H, 1), lambda i: (0, 0)),
            pl.BlockSpec((1, 1), lambda i: (0, 0)),
            pl.BlockSpec((_H, _C), lambda i: (0, 0)),
            pl.BlockSpec((1, _C), lambda i: (0, 0)),
        ],
        out_specs=pl.BlockSpec((1, _C), lambda i: (0, 0)),
        out_shape=jax.ShapeDtypeStruct((1, _C), jnp.float32),
        scratch_shapes=[
            pltpu.SMEM((1, 1), jnp.float32),
            pltpu.SMEM((1, 1), jnp.float32),
            pltpu.VMEM((1, _H), jnp.float32),
        ],
    )(p0, p1, wn, bn, wg, bg, wf, bf)


def kernel(x, edge_index, W_edge, b_edge, W_node, b_node, W_gate, b_gate,
           W_fc, b_fc):
    xp = jnp.pad(x, ((0, _NP - _N), (0, 0)))
    xa, bt = _make_tables(xp, W_edge[:_D], W_edge[_D:],
                          b_edge.reshape(1, _H))

    e = edge_index.shape[1]
    zrows = _N + jnp.arange(_EP - e, dtype=jnp.int32) % (_NP - _N)
    src = jnp.concatenate(
        [edge_index[0], zrows]).reshape(_NW * _CHUNKS, _K)
    dump = jnp.arange(_EP - e, dtype=jnp.int32) % _N
    dst = jnp.concatenate([edge_index[1], dump]).reshape(_NW * _CHUNKS, _K)

    partials = _sc_edges(xa, bt, src, dst)
    partials = jnp.pad(partials, ((0, 0), (0, _NP - _NPS), (0, 0)))

    return _readout(partials[0], partials[1], W_node,
                    b_node.reshape(1, _H), W_gate, b_gate.reshape(1, 1),
                    W_fc, b_fc.reshape(1, _C))


# speedup vs baseline: 2.1528x; 1.0073x over previous
import functools

import jax
import jax.numpy as jnp
from jax import lax
from jax.experimental import pallas as pl
from jax.experimental.pallas import tpu as pltpu
from jax.experimental.pallas import tpu_sc as plsc

_N = 10000
_D = 128
_H = 128
_C = 10

_NP = 10240
_NPS = 10000
_K = 40
_RING = 3
_NW = 32
_CHUNKS = 256
_IBLK = 8
_NBLK = _CHUNKS // _IBLK
_EPT = _CHUNKS * _K
_EP = _NW * _EPT
_RPTA = 632
_RPTB = _NPS - 15 * _RPTA


def _tables_body(x_ref, w1_ref, w2_ref, be_ref, xa_ref, bt_ref):
    xb = x_ref[...]
    xa_ref[:, 0:_D] = xb
    xa_ref[:, _D:2 * _D] = jnp.dot(xb, w1_ref[...],
                                   preferred_element_type=jnp.float32)
    bt_ref[...] = jnp.dot(xb, w2_ref[...],
                          preferred_element_type=jnp.float32) + be_ref[...]


def _make_tables(xp, w1, w2, be):
    blk = 1024
    grid = _NP // blk
    return pl.pallas_call(
        _tables_body,
        grid=(grid,),
        in_specs=[
            pl.BlockSpec((blk, _D), lambda i: (i, 0)),
            pl.BlockSpec((_D, _H), lambda i: (0, 0)),
            pl.BlockSpec((_D, _H), lambda i: (0, 0)),
            pl.BlockSpec((1, _H), lambda i: (0, 0)),
        ],
        out_specs=[
            pl.BlockSpec((blk, 2 * _D), lambda i: (i, 0)),
            pl.BlockSpec((blk, _H), lambda i: (i, 0)),
        ],
        out_shape=[
            jax.ShapeDtypeStruct((_NP, 2 * _D), jnp.float32),
            jax.ShapeDtypeStruct((_NP, _H), jnp.float32),
        ],
    )(xp, w1, w2, be)


def _sc_body(xa_hbm, bt_hbm, src_hbm, dst_hbm, out_hbm,
             src_i, dst_i, xa0, xa1, xa2, b0, b1, b2, hneigh,
             sx0, sx1, sx2, sb0, sb1, sb2, ss0, ss1, ss2):
    c = lax.axis_index("c")
    s = lax.axis_index("s")
    wid = c * 16 + s
    rbase = wid * _CHUNKS

    bufs = ((xa0, b0, sx0, sb0, ss0),
            (xa1, b1, sx1, sb1, ss1),
            (xa2, b2, sx2, sb2, ss2))

    def _zrow(r, carry):
        for cc in range(8):
            b0[r, pl.ds(cc * 16, 16)] = jnp.zeros((16,), jnp.float32)
        return carry
    lax.fori_loop(0, _K, _zrow, 0)

    def _span(body):
        @pl.when(s < 15)
        def _():
            row0 = pl.multiple_of(s * _RPTA, 8)
            for j in range(_RPTA // _K):
                body(row0 + j * _K, _K)
            body(row0 + (_RPTA // _K) * _K, _RPTA % _K)

        @pl.when(s == 15)
        def _():
            row0 = 15 * _RPTA
            for j in range(_RPTB // _K):
                body(row0 + j * _K, _K)

    def _zcopy(r0, n):
        pltpu.sync_copy(b0.at[pl.ds(0, n)], hneigh.at[pl.ds(r0, n)])
    _span(_zcopy)
    plsc.subcore_barrier()

    def _load_iblock(b):
        r0 = pl.multiple_of(rbase + b * _IBLK, 8)
        q = b % 2
        pltpu.sync_copy(src_hbm.at[pl.ds(r0, _IBLK)], src_i.at[q])
        pltpu.sync_copy(dst_hbm.at[pl.ds(r0, _IBLK)], dst_i.at[q])

    def _issue(i, p):
        xa, bb, sx, sb, ss = bufs[p]
        q = (i // _IBLK) % 2
        j = i % _IBLK
        pltpu.async_copy(xa_hbm.at[src_i.at[q, j]], xa, sx)
        pltpu.async_copy(bt_hbm.at[dst_i.at[q, j]], bb, sb)

    def _process(i, p):
        xa, bb, sx, sb, ss = bufs[p]
        q = (i // _IBLK) % 2
        j = i % _IBLK

        pltpu.make_async_copy(xa_hbm.at[src_i.at[q, j]], xa, sx).wait()
        pltpu.make_async_copy(bt_hbm.at[dst_i.at[q, j]], bb, sb).wait()

        @plsc.parallel_loop(0, _K, unroll=4)
        def _crow(r):
            for cc in range(8):
                xv = xa[r, pl.ds(cc * 16, 16)]
                av = xa[r, pl.ds(_D + cc * 16, 16)]
                bv = bb[r, pl.ds(cc * 16, 16)]
                bb[r, pl.ds(cc * 16, 16)] = xv * jnp.maximum(av + bv, 0.0)

        pltpu.async_copy(bb, hneigh.at[dst_i.at[q, j]], ss, add=True)

        pn = (p + 2) % _RING
        bbn = bufs[pn][1]
        ssn = bufs[pn][4]

        @pl.when(i >= 1)
        def _():
            pltpu.make_async_copy(bbn, hneigh.at[dst_i.at[q, j]], ssn).wait()

        @pl.when((i % _IBLK == 0) & (i // _IBLK + 1 < _NBLK))
        def _():
            _load_iblock(i // _IBLK + 1)

        @pl.when(i + 2 < _CHUNKS)
        def _():
            _issue(i + 2, pn)

    _load_iblock(0)
    _issue(0, 0)
    _issue(1, 1)

    def _step(k, carry):
        _process(3 * k, 0)
        _process(3 * k + 1, 1)
        _process(3 * k + 2, 2)
        return carry
    lax.fori_loop(0, _CHUNKS // 3, _step, 0)
    _process(jnp.int32(_CHUNKS - 1), (_CHUNKS - 1) % _RING)

    lastb = bufs[(_CHUNKS - 1) % _RING]
    pltpu.make_async_copy(lastb[1], hneigh.at[dst_i.at[0, 0]],
                          lastb[4]).wait()

    plsc.subcore_barrier()

    def _dcopy(r0, n):
        pltpu.sync_copy(hneigh.at[pl.ds(r0, n)], b0.at[pl.ds(0, n)])
        pltpu.sync_copy(b0.at[pl.ds(0, n)], out_hbm.at[c, pl.ds(r0, n)])
    _span(_dcopy)


_sc_edges = functools.partial(
    pl.kernel,
    out_type=jax.ShapeDtypeStruct((2, _NPS, _H), jnp.float32),
    mesh=plsc.VectorSubcoreMesh(core_axis_name="c", subcore_axis_name="s"),
    scratch_types=(
        [pltpu.VMEM((2, _IBLK, _K), jnp.int32)] * 2
        + [pltpu.VMEM((_K, 2 * _D), jnp.float32)] * _RING
        + [pltpu.VMEM((_K, _H), jnp.float32)] * _RING
        + [pltpu.VMEM_SHARED((_NPS, _H), jnp.float32)]
        + [pltpu.SemaphoreType.DMA] * (3 * _RING)
    ),
)(_sc_body)


def _readout_body(p0_ref, p1_ref, wn_ref, bn_ref, wg_ref, bg_ref,
                  wf_ref, bf_ref, out_ref, m_ref, s_ref, r_ref):
    step = pl.program_id(0)

    @pl.when(step == 0)
    def _():
        m_ref[0, 0] = -1e30
        s_ref[0, 0] = 0.0
        r_ref[...] = jnp.zeros_like(r_ref)

    p = p0_ref[...] + p1_ref[...]
    h = jnp.maximum(
        jnp.dot(p, wn_ref[...], preferred_element_type=jnp.float32)
        + bn_ref[...], 0.0)
    g = jnp.dot(h, wg_ref[...], preferred_element_type=jnp.float32) + bg_ref[0, 0]
    rows = step * p.shape[0] + lax.broadcasted_iota(jnp.int32, g.shape, 0)
    g = jnp.where(rows < _N, g, -1e30)

    m_old = m_ref[0, 0]
    m_new = jnp.maximum(m_old, jnp.max(g))
    scale = jnp.exp(m_old - m_new)
    e = jnp.exp(g - m_new)
    s_ref[0, 0] = s_ref[0, 0] * scale + jnp.sum(e)
    r_ref[...] = r_ref[...] * scale + jnp.sum(e * h, axis=0, keepdims=True)
    m_ref[0, 0] = m_new

    @pl.when(step == pl.num_programs(0) - 1)
    def _():
        readout = r_ref[...] / s_ref[0, 0]
        out_ref[...] = jnp.dot(readout, wf_ref[...],
                               preferred_element_type=jnp.float32) + bf_ref[...]


def _readout(p0, p1, wn, bn, wg, bg, wf, bf):
    blk = 1024
    grid = _NP // blk
    return pl.pallas_call(
        _readout_body,
        grid=(grid,),
        in_specs=[
            pl.BlockSpec((blk, _H), lambda i: (i, 0)),
            pl.BlockSpec((blk, _H), lambda i: (i, 0)),
            pl.BlockSpec((_H, _H), lambda i: (0, 0)),
            pl.BlockSpec((1, _H), lambda i: (0, 0)),
            pl.BlockSpec((_H, 1), lambda i: (0, 0)),
            pl.BlockSpec((1, 1), lambda i: (0, 0)),
            pl.BlockSpec((_H, _C), lambda i: (0, 0)),
            pl.BlockSpec((1, _C), lambda i: (0, 0)),
        ],
        out_specs=pl.BlockSpec((1, _C), lambda i: (0, 0)),
        out_shape=jax.ShapeDtypeStruct((1, _C), jnp.float32),
        scratch_shapes=[
            pltpu.SMEM((1, 1), jnp.float32),
            pltpu.SMEM((1, 1), jnp.float32),
            pltpu.VMEM((1, _H), jnp.float32),
        ],
    )(p0, p1, wn, bn, wg, bg, wf, bf)


def kernel(x, edge_index, W_edge, b_edge, W_node, b_node, W_gate, b_gate,
           W_fc, b_fc):
    xp = jnp.pad(x, ((0, _NP - _N), (0, 0)))
    xa, bt = _make_tables(xp, W_edge[:_D], W_edge[_D:],
                          b_edge.reshape(1, _H))

    e = edge_index.shape[1]
    zrows = _N + jnp.arange(_EP - e, dtype=jnp.int32) % (_NP - _N)
    src = jnp.concatenate(
        [edge_index[0], zrows]).reshape(_NW * _CHUNKS, _K)
    dump = jnp.arange(_EP - e, dtype=jnp.int32) % _N
    dst = jnp.concatenate([edge_index[1], dump]).reshape(_NW * _CHUNKS, _K)

    partials = _sc_edges(xa, bt, src, dst)
    partials = jnp.pad(partials, ((0, 0), (0, _NP - _NPS), (0, 0)))

    return _readout(partials[0], partials[1], W_node,
                    b_node.reshape(1, _H), W_gate, b_gate.reshape(1, 1),
                    W_fc, b_fc.reshape(1, _C))


# speedup vs baseline: 2.1568x; 1.0019x over previous
import functools

import jax
import jax.numpy as jnp
from jax import lax
from jax.experimental import pallas as pl
from jax.experimental.pallas import tpu as pltpu
from jax.experimental.pallas import tpu_sc as plsc

_N = 10000
_D = 128
_H = 128
_C = 10

_NP = 10240
_NPS = 10000
_K = 40
_RING = 3
_NW = 32
_CHUNKS = 256
_IBLK = 8
_NBLK = _CHUNKS // _IBLK
_EPT = _CHUNKS * _K
_EP = _NW * _EPT
_RPTA = 632
_RPTB = _NPS - 15 * _RPTA


def _tables_body(x_ref, w1_ref, w2_ref, be_ref, xa_ref, bt_ref):
    xb = x_ref[...]
    xa_ref[:, 0:_D] = xb
    xa_ref[:, _D:2 * _D] = jnp.dot(xb, w1_ref[...],
                                   preferred_element_type=jnp.float32)
    bt_ref[...] = jnp.dot(xb, w2_ref[...],
                          preferred_element_type=jnp.float32) + be_ref[...]


def _make_tables(xp, w1, w2, be):
    blk = 1024
    grid = _NP // blk
    return pl.pallas_call(
        _tables_body,
        grid=(grid,),
        in_specs=[
            pl.BlockSpec((blk, _D), lambda i: (i, 0)),
            pl.BlockSpec((_D, _H), lambda i: (0, 0)),
            pl.BlockSpec((_D, _H), lambda i: (0, 0)),
            pl.BlockSpec((1, _H), lambda i: (0, 0)),
        ],
        out_specs=[
            pl.BlockSpec((blk, 2 * _D), lambda i: (i, 0)),
            pl.BlockSpec((blk, _H), lambda i: (i, 0)),
        ],
        out_shape=[
            jax.ShapeDtypeStruct((_NP, 2 * _D), jnp.float32),
            jax.ShapeDtypeStruct((_NP, _H), jnp.float32),
        ],
    )(xp, w1, w2, be)


def _sc_body(xa_hbm, bt_hbm, src_hbm, dst_hbm, out_hbm,
             src_i, dst_i, xa0, xa1, xa2, b0, b1, b2, hneigh,
             sx0, sx1, sx2, sb0, sb1, sb2, ss0, ss1, ss2):
    c = lax.axis_index("c")
    s = lax.axis_index("s")
    wid = c * 16 + s
    rbase = wid * _CHUNKS

    bufs = ((xa0, b0, sx0, sb0, ss0),
            (xa1, b1, sx1, sb1, ss1),
            (xa2, b2, sx2, sb2, ss2))

    def _zrow(r, carry):
        for cc in range(8):
            b0[r, pl.ds(cc * 16, 16)] = jnp.zeros((16,), jnp.float32)
        return carry
    lax.fori_loop(0, _K, _zrow, 0)

    def _span(body):
        @pl.when(s < 15)
        def _():
            row0 = pl.multiple_of(s * _RPTA, 8)
            for j in range(_RPTA // _K):
                body(row0 + j * _K, _K)
            body(row0 + (_RPTA // _K) * _K, _RPTA % _K)

        @pl.when(s == 15)
        def _():
            row0 = 15 * _RPTA
            for j in range(_RPTB // _K):
                body(row0 + j * _K, _K)

    def _zcopy(r0, n):
        pltpu.sync_copy(b0.at[pl.ds(0, n)], hneigh.at[pl.ds(r0, n)])
    _span(_zcopy)
    plsc.subcore_barrier()

    def _load_iblock(b):
        r0 = pl.multiple_of(rbase + b * _IBLK, 8)
        q = b % 2
        pltpu.sync_copy(src_hbm.at[pl.ds(r0, _IBLK)], src_i.at[q])
        pltpu.sync_copy(dst_hbm.at[pl.ds(r0, _IBLK)], dst_i.at[q])

    def _issue(i, p):
        xa, bb, sx, sb, ss = bufs[p]
        q = (i // _IBLK) % 2
        j = i % _IBLK
        pltpu.async_copy(xa_hbm.at[src_i.at[q, j]], xa, sx)
        pltpu.async_copy(bt_hbm.at[dst_i.at[q, j]], bb, sb)

    def _process(i, p):
        xa, bb, sx, sb, ss = bufs[p]
        q = (i // _IBLK) % 2
        j = i % _IBLK

        pltpu.make_async_copy(xa_hbm.at[src_i.at[q, j]], xa, sx).wait()
        pltpu.make_async_copy(bt_hbm.at[dst_i.at[q, j]], bb, sb).wait()

        @plsc.parallel_loop(0, _K, unroll=4)
        def _crow(r):
            for cc in range(8):
                xv = xa[r, pl.ds(cc * 16, 16)]
                av = xa[r, pl.ds(_D + cc * 16, 16)]
                bv = bb[r, pl.ds(cc * 16, 16)]
                bb[r, pl.ds(cc * 16, 16)] = xv * jnp.maximum(av + bv, 0.0)

        pltpu.async_copy(bb, hneigh.at[dst_i.at[q, j]], ss, add=True)

        pn = (p + 2) % _RING
        bbn = bufs[pn][1]
        ssn = bufs[pn][4]

        @pl.when(i >= 1)
        def _():
            pltpu.make_async_copy(bbn, hneigh.at[dst_i.at[q, j]], ssn).wait()

        @pl.when((i % _IBLK == 0) & (i // _IBLK + 1 < _NBLK))
        def _():
            _load_iblock(i // _IBLK + 1)

        @pl.when(i + 2 < _CHUNKS)
        def _():
            _issue(i + 2, pn)

    _load_iblock(0)
    _issue(0, 0)
    _issue(1, 1)

    def _step(k, carry):
        _process(3 * k, 0)
        _process(3 * k + 1, 1)
        _process(3 * k + 2, 2)
        return carry
    lax.fori_loop(0, _CHUNKS // 3, _step, 0)
    _process(jnp.int32(_CHUNKS - 1), (_CHUNKS - 1) % _RING)

    lastb = bufs[(_CHUNKS - 1) % _RING]
    pltpu.make_async_copy(lastb[1], hneigh.at[dst_i.at[0, 0]],
                          lastb[4]).wait()

    plsc.subcore_barrier()

    def _dcopy(r0, n):
        pltpu.sync_copy(hneigh.at[pl.ds(r0, n)], b0.at[pl.ds(0, n)])
        pltpu.sync_copy(b0.at[pl.ds(0, n)], out_hbm.at[c, pl.ds(r0, n)])
    _span(_dcopy)


_sc_edges = functools.partial(
    pl.kernel,
    out_type=jax.ShapeDtypeStruct((2, _NP, _H), jnp.float32),
    mesh=plsc.VectorSubcoreMesh(core_axis_name="c", subcore_axis_name="s"),
    scratch_types=(
        [pltpu.VMEM((2, _IBLK, _K), jnp.int32)] * 2
        + [pltpu.VMEM((_K, 2 * _D), jnp.float32)] * _RING
        + [pltpu.VMEM((_K, _H), jnp.float32)] * _RING
        + [pltpu.VMEM_SHARED((_NPS, _H), jnp.float32)]
        + [pltpu.SemaphoreType.DMA] * (3 * _RING)
    ),
)(_sc_body)


def _readout_body(p0_ref, p1_ref, wn_ref, bn_ref, wg_ref, bg_ref,
                  wf_ref, bf_ref, out_ref, m_ref, s_ref, r_ref):
    step = pl.program_id(0)

    @pl.when(step == 0)
    def _():
        m_ref[0, 0] = -1e30
        s_ref[0, 0] = 0.0
        r_ref[...] = jnp.zeros_like(r_ref)

    p = p0_ref[...] + p1_ref[...]
    h = jnp.maximum(
        jnp.dot(p, wn_ref[...], preferred_element_type=jnp.float32)
        + bn_ref[...], 0.0)
    g = jnp.dot(h, wg_ref[...], preferred_element_type=jnp.float32) + bg_ref[0, 0]
    rows = step * p.shape[0] + lax.broadcasted_iota(jnp.int32, g.shape, 0)
    g = jnp.where(rows < _N, g, -1e30)
    h = jnp.where(rows < _N, h, 0.0)

    m_old = m_ref[0, 0]
    m_new = jnp.maximum(m_old, jnp.max(g))
    scale = jnp.exp(m_old - m_new)
    e = jnp.exp(g - m_new)
    s_ref[0, 0] = s_ref[0, 0] * scale + jnp.sum(e)
    r_ref[...] = r_ref[...] * scale + jnp.sum(e * h, axis=0, keepdims=True)
    m_ref[0, 0] = m_new

    @pl.when(step == pl.num_programs(0) - 1)
    def _():
        readout = r_ref[...] / s_ref[0, 0]
        out_ref[...] = jnp.dot(readout, wf_ref[...],
                               preferred_element_type=jnp.float32) + bf_ref[...]


def _readout(p0, p1, wn, bn, wg, bg, wf, bf):
    blk = 1024
    grid = _NP // blk
    return pl.pallas_call(
        _readout_body,
        grid=(grid,),
        in_specs=[
            pl.BlockSpec((blk, _H), lambda i: (i, 0)),
            pl.BlockSpec((blk, _H), lambda i: (i, 0)),
            pl.BlockSpec((_H, _H), lambda i: (0, 0)),
            pl.BlockSpec((1, _H), lambda i: (0, 0)),
            pl.BlockSpec((_H, 1), lambda i: (0, 0)),
            pl.BlockSpec((1, 1), lambda i: (0, 0)),
            pl.BlockSpec((_H, _C), lambda i: (0, 0)),
            pl.BlockSpec((1, _C), lambda i: (0, 0)),
        ],
        out_specs=pl.BlockSpec((1, _C), lambda i: (0, 0)),
        out_shape=jax.ShapeDtypeStruct((1, _C), jnp.float32),
        scratch_shapes=[
            pltpu.SMEM((1, 1), jnp.float32),
            pltpu.SMEM((1, 1), jnp.float32),
            pltpu.VMEM((1, _H), jnp.float32),
        ],
    )(p0, p1, wn, bn, wg, bg, wf, bf)


def kernel(x, edge_index, W_edge, b_edge, W_node, b_node, W_gate, b_gate,
           W_fc, b_fc):
    xp = jnp.pad(x, ((0, _NP - _N), (0, 0)))
    xa, bt = _make_tables(xp, W_edge[:_D], W_edge[_D:],
                          b_edge.reshape(1, _H))

    e = edge_index.shape[1]
    zrows = _N + jnp.arange(_EP - e, dtype=jnp.int32) % (_NP - _N)
    src = jnp.concatenate(
        [edge_index[0], zrows]).reshape(_NW * _CHUNKS, _K)
    dump = jnp.arange(_EP - e, dtype=jnp.int32) % _N
    dst = jnp.concatenate([edge_index[1], dump]).reshape(_NW * _CHUNKS, _K)

    partials = _sc_edges(xa, bt, src, dst)

    return _readout(partials[0], partials[1], W_node,
                    b_node.reshape(1, _H), W_gate, b_gate.reshape(1, 1),
                    W_fc, b_fc.reshape(1, _C))
